# Initial kernel scaffold; baseline (speedup 1.0000x reference)
import jax
import jax.numpy as jnp
from jax.experimental import pallas as pl


def kernel(W1, W3, s, r, y, p):
    raise NotImplementedError("write your pallas kernel here")


# speedup vs baseline: 1.1305x; 1.1305x over previous
import functools

import jax
import jax.numpy as jnp
from jax import lax
from jax.experimental import pallas as pl
from jax.experimental.pallas import tpu as pltpu
from jax.experimental.pallas import tpu_sc as plsc

L = 12
H = 16
HS = 64
IN1 = 1024
IN3 = 4096
_TOP = 0x7FFFFFFF


def _sq_reduce_body(w1_ref, w3_ref, colsq_ref, sc3_ref):
    x1 = w1_ref[0]
    colsq_ref[0, 0, :] = jnp.sum(x1 * x1, axis=0)
    x3 = w3_ref[0]
    sc3_ref[0, 0, :] = jnp.sum(x3 * x3, axis=0)


def _tc_reduce(W1, W3):
    return pl.pallas_call(
        _sq_reduce_body,
        grid=(L,),
        in_specs=[
            pl.BlockSpec((1, IN1, IN1), lambda l: (l, 0, 0)),
            pl.BlockSpec((1, IN1, IN3), lambda l: (l, 0, 0)),
        ],
        out_specs=[
            pl.BlockSpec((1, 1, IN1), lambda l: (l, 0, 0)),
            pl.BlockSpec((1, 1, IN3), lambda l: (l, 0, 0)),
        ],
        out_shape=[
            jax.ShapeDtypeStruct((L, 1, IN1), jnp.float32),
            jax.ShapeDtypeStruct((L, 1, IN3), jnp.float32),
        ],
    )(W1, W3)


def _sc_body(colsq_hbm, sc3_hbm, pi_hbm, pf_hbm, out_hbm,
             buf4k, buf1k, pi_v, pf_v, outv):
    cid = lax.axis_index("c")
    sid = lax.axis_index("s")
    wid = cid * 16 + sid
    lane = lax.iota(jnp.int32, 16)
    zeros_f = jnp.zeros((16,), jnp.float32)
    zeros_i = jnp.zeros((16,), jnp.int32)
    top_v = jnp.full((16,), _TOP, jnp.int32)
    outv[...] = zeros_f

    @pl.when((cid == 0) & (sid < L))
    def _big():
        pltpu.sync_copy(sc3_hbm.at[sid], buf4k)
        pltpu.sync_copy(pi_hbm.at[sid], pi_v)
        pltpu.sync_copy(pf_hbm.at[sid], pf_v)
        k2 = pi_v[pl.ds(16, 16)][1]
        y1 = pf_v[pl.ds(16, 16)][1]
        kvec = jnp.full((16,), 1, jnp.int32) * k2

        @plsc.parallel_loop(0, IN3 // 16, unroll=8,
                            carry=(zeros_f, zeros_i,
                                   jnp.full((16,), _TOP, jnp.int32)))
        def _init(i, c):
            tv, mx, mn = c
            wb = buf4k[pl.ds(i * 16, 16)]
            return (tv + lax.bitcast_convert_type(wb, jnp.float32),
                    jnp.maximum(mx, wb), jnp.minimum(mn, wb))

        totv, mxv, mnv = _init
        tot_s = jnp.sum(totv)
        lo0 = jnp.full((16,), 1, jnp.int32) * jnp.min(mnv)
        hi0 = jnp.full((16,), 1, jnp.int32) * jnp.max(mxv)

        def unconverged(carry):
            lo, hi = carry
            return (hi - lo)[0] > 0

        def qpass(carry):
            lo, hi = carry
            mid = lo + lax.shift_right_logical(hi - lo, 1)

            @plsc.parallel_loop(0, IN3 // 16, unroll=8, carry=zeros_i)
            def _cnt(i, cnt):
                wb = buf4k[pl.ds(i * 16, 16)]
                return cnt + plsc.all_reduce_population_count(wb <= mid)

            ge = _cnt >= kvec
            return jnp.where(ge, lo, mid + 1), jnp.where(ge, mid, hi)

        lo, _ = lax.while_loop(unconverged, qpass, (lo0, hi0))
        t = lax.bitcast_convert_type(lo, jnp.float32)

        @plsc.parallel_loop(0, IN3 // 16, unroll=8, carry=(zeros_i, zeros_f))
        def _fin(i, c):
            clt, slt = c
            wb = buf4k[pl.ds(i * 16, 16)]
            m = wb < lo
            return (clt + plsc.all_reduce_population_count(m),
                    slt + jnp.where(
                        m, lax.bitcast_convert_type(wb, jnp.float32), 0.0))

        cltv, sltv = _fin
        res = jnp.where(
            k2 >= IN3, tot_s,
            jnp.where(k2 <= 0, 0.0,
                      jnp.sum(sltv)
                      + (k2 - jnp.max(cltv)).astype(jnp.float32)
                      * jnp.max(t)))
        outv[...] = jnp.where(lane == 0, y1 * res, 0.0)

    @pl.when((cid == 1) & (sid < L))
    def _small():
        pltpu.sync_copy(colsq_hbm.at[sid], buf1k)
        pltpu.sync_copy(pi_hbm.at[sid], pi_v)
        pltpu.sync_copy(pf_hbm.at[sid], pf_v)
        kr = pi_v[pl.ds(0, 16)]
        pvec = pf_v[pl.ds(0, 16)]
        k1 = pi_v[pl.ds(16, 16)][0]
        y0 = pf_v[pl.ds(16, 16)][0]
        base = lane * HS

        @plsc.parallel_loop(0, HS, unroll=8, carry=(zeros_f, zeros_i, top_v))
        def _init(j, c):
            tv, mx, mn = c
            wb = plsc.load_gather(buf1k, [base + j])
            return (tv + lax.bitcast_convert_type(wb, jnp.float32),
                    jnp.maximum(mx, wb), jnp.minimum(mn, wb))

        tot, mxv, mnv = _init

        def unconverged(carry):
            lo, hi = carry
            return jnp.max(hi - lo) > 0

        def qpass(carry):
            lo, hi = carry
            mid = lo + lax.shift_right_logical(hi - lo, 1)

            @plsc.parallel_loop(0, HS, unroll=8, carry=zeros_i)
            def _cnt(j, cnt):
                wb = plsc.load_gather(buf1k, [base + j])
                return cnt + jnp.where(wb <= mid, 1, 0).astype(jnp.int32)

            ge = _cnt >= kr
            return jnp.where(ge, lo, mid + 1), jnp.where(ge, mid, hi)

        lo, _ = lax.while_loop(unconverged, qpass, (mnv, mxv))
        t = lax.bitcast_convert_type(lo, jnp.float32)

        @plsc.parallel_loop(0, HS, unroll=8, carry=(zeros_i, zeros_f))
        def _fin(j, c):
            clt, slt = c
            wb = plsc.load_gather(buf1k, [base + j])
            m = wb < lo
            return (clt + jnp.where(m, 1, 0).astype(jnp.int32),
                    slt + jnp.where(
                        m, lax.bitcast_convert_type(wb, jnp.float32), 0.0))

        cltv, sltv = _fin
        res = jnp.where(
            kr >= HS, tot,
            jnp.where(kr <= 0, 0.0,
                      sltv + (kr - cltv).astype(jnp.float32) * t))
        rres = jnp.sum(pvec * res)
        sk, _ = plsc.sort_key_val(tot, tot)
        a_val = jnp.sum(jnp.where(lane < k1, sk, 0.0))
        outv[...] = jnp.where(lane == 0, y0 * a_val + rres, 0.0)

    pltpu.sync_copy(outv, out_hbm.at[wid])


def _sc_select(colsq, sc3, pi, pf):
    mesh = plsc.VectorSubcoreMesh(core_axis_name="c", subcore_axis_name="s")
    run = functools.partial(
        pl.kernel,
        mesh=mesh,
        compiler_params=pltpu.CompilerParams(needs_layout_passes=False),
        out_type=jax.ShapeDtypeStruct((32, 16), jnp.float32),
        scratch_types=[
            pltpu.VMEM((IN3,), jnp.int32),
            pltpu.VMEM((IN1,), jnp.int32),
            pltpu.VMEM((32,), jnp.int32),
            pltpu.VMEM((32,), jnp.float32),
            pltpu.VMEM((16,), jnp.float32),
        ],
    )(_sc_body)
    return run(colsq, sc3, pi, pf)


def kernel(W1, W3, s, r, y, p):
    colsq3, sc33 = _tc_reduce(W1, W3)
    colsq = lax.bitcast_convert_type(colsq3.reshape(L, IN1), jnp.int32)
    sc3 = lax.bitcast_convert_type(sc33.reshape(L, IN3), jnp.int32)
    k1 = jnp.ceil(s[:, 0]).astype(jnp.int32)
    k2 = jnp.ceil(s[:, 1]).astype(jnp.int32)
    kr = jnp.ceil(r).astype(jnp.int32)
    pi = jnp.concatenate(
        [kr, k1[:, None], k2[:, None], jnp.zeros((L, 14), jnp.int32)], axis=1)
    pf = jnp.concatenate([p, y, jnp.zeros((L, 14), jnp.float32)], axis=1)
    parts = _sc_select(colsq, sc3, pi, pf)
    return jnp.sum(parts)
